# Initial kernel scaffold; baseline (speedup 1.0000x reference)
import jax
import jax.numpy as jnp
from jax.experimental import pallas as pl


def kernel(A_base, edge_index, edge_gates):
    raise NotImplementedError("write your pallas kernel here")


# speedup vs baseline: 3.2427x; 3.2427x over previous
import functools

import jax
import jax.numpy as jnp
from jax import lax
from jax.experimental import pallas as pl
from jax.experimental.pallas import tpu as pltpu
from jax.experimental.pallas import tpu_sc as plsc

_ALPHA = 0.005
_N = 2048
_E = 65536
_B = 8

_NC = 2
_NS = 16
_CHUNK_ROWS = 512
_NCHUNK = _N // _CHUNK_ROWS
_CHUNK_ELEMS = _CHUNK_ROWS * _N
_CHUNK_SHIFT = 20
_CHUNK_MASK = _CHUNK_ELEMS - 1
_SLICE = _CHUNK_ELEMS // _NS
_EPT = _E // _NS
_ROWS = _EPT // 128
_BPC = _B // _NC


def _body(a_hbm, src_hbm, dst_hbm, gates_hbm, out_hbm,
          srcv, dstv, flatv, gv, wv, idx2, upd2, shared):
  c = lax.axis_index("c")
  s = lax.axis_index("s")
  ebase = s * _EPT

  pltpu.sync_copy(src_hbm.at[pl.ds(ebase, _EPT)], srcv)
  pltpu.sync_copy(dst_hbm.at[pl.ds(ebase, _EPT)], dstv)

  def init_row(j, carry):
    for q in range(8):
      sl = pl.ds(j * 128 + q * 16, 16)
      v = srcv[sl] * _N + dstv[sl]
      flatv[sl] = v
      idx2[j, pl.ds(q * 16, 16)] = v & _CHUNK_MASK
    return carry
  lax.fori_loop(0, _ROWS, init_row, 0)

  for i in range(_BPC):
    b = _NC * i + c
    pltpu.sync_copy(gates_hbm.at[b, pl.ds(ebase, _EPT)], gv)

    def wrow(j, carry):
      for q in range(8):
        sl = pl.ds(j * 128 + q * 16, 16)
        x = gv[sl]
        x = jnp.minimum(jnp.maximum(x, -3.0), 3.0)
        wv[sl] = _ALPHA / (1.0 + jnp.exp(-x))
      return carry
    lax.fori_loop(0, _ROWS, wrow, 0)

    for k in range(_NCHUNK):
      pltpu.sync_copy(
          a_hbm.at[pl.ds(k * _CHUNK_ELEMS + s * _SLICE, _SLICE)],
          shared.at[pl.ds(s * _SLICE, _SLICE)])
      plsc.subcore_barrier()

      def srow(j, carry):
        for q in range(8):
          sl = pl.ds(j * 128 + q * 16, 16)
          in_chunk = (flatv[sl] >> _CHUNK_SHIFT) == k
          upd2[j, pl.ds(q * 16, 16)] = jnp.where(in_chunk, wv[sl], 0.0)
        pltpu.sync_copy(upd2.at[j], shared.at[idx2.at[j]], add=True)
        return carry
      lax.fori_loop(0, _ROWS, srow, 0)
      plsc.subcore_barrier()

      pltpu.sync_copy(
          shared.at[pl.ds(s * _SLICE, _SLICE)],
          out_hbm.at[b, pl.ds(k * _CHUNK_ELEMS + s * _SLICE, _SLICE)])
      plsc.subcore_barrier()


@jax.jit
def _dyn_adj(a_flat, src, dst, gates):
  mesh = plsc.VectorSubcoreMesh(core_axis_name="c", subcore_axis_name="s")
  f = functools.partial(
      pl.kernel,
      out_type=jax.ShapeDtypeStruct((_B, _N * _N), jnp.float32),
      mesh=mesh,
      scratch_types=[
          pltpu.VMEM((_EPT,), jnp.int32),
          pltpu.VMEM((_EPT,), jnp.int32),
          pltpu.VMEM((_EPT,), jnp.int32),
          pltpu.VMEM((_EPT,), jnp.float32),
          pltpu.VMEM((_EPT,), jnp.float32),
          pltpu.VMEM((_ROWS, 128), jnp.int32),
          pltpu.VMEM((_ROWS, 128), jnp.float32),
          pltpu.VMEM_SHARED((_CHUNK_ELEMS,), jnp.float32),
      ],
  )(_body)
  return f(a_flat, src, dst, gates)


def kernel(A_base, edge_index, edge_gates):
  ei = edge_index.astype(jnp.int32)
  out = _dyn_adj(A_base.reshape(_N * _N), ei[0], ei[1], edge_gates)
  return out.reshape(_B, _N, _N)


# speedup vs baseline: 3.5653x; 1.0995x over previous
import functools

import jax
import jax.numpy as jnp
from jax import lax
from jax.experimental import pallas as pl
from jax.experimental.pallas import tpu as pltpu
from jax.experimental.pallas import tpu_sc as plsc

_ALPHA = 0.005
_N = 2048
_E = 65536
_B = 8

_NC = 2
_NS = 16
_CHUNK_ROWS = 512
_NCHUNK = _N // _CHUNK_ROWS
_CHUNK_ELEMS = _CHUNK_ROWS * _N
_CHUNK_SHIFT = 20
_CHUNK_MASK = _CHUNK_ELEMS - 1
_SLICE = _CHUNK_ELEMS // _NS
_EPT = _E // _NS
_ROWS = _EPT // 128
_BPC = _B // _NC


def _body(a_hbm, src_hbm, dst_hbm, gates_hbm, out_hbm,
          srcv, dstv, flatv, gv, wv, idx2, upd2, shared, sem):
  c = lax.axis_index("c")
  s = lax.axis_index("s")
  ebase = s * _EPT

  pltpu.sync_copy(src_hbm.at[pl.ds(ebase, _EPT)], srcv)
  pltpu.sync_copy(dst_hbm.at[pl.ds(ebase, _EPT)], dstv)

  def init_row(j, carry):
    for q in range(8):
      sl = pl.ds(j * 128 + q * 16, 16)
      v = srcv[sl] * _N + dstv[sl]
      flatv[sl] = v
      idx2[j, pl.ds(q * 16, 16)] = v & _CHUNK_MASK
    return carry
  lax.fori_loop(0, _ROWS, init_row, 0)

  for i in range(_BPC):
    b = _NC * i + c
    pltpu.sync_copy(gates_hbm.at[b, pl.ds(ebase, _EPT)], gv)

    def wrow(j, carry):
      for q in range(8):
        sl = pl.ds(j * 128 + q * 16, 16)
        x = gv[sl]
        x = jnp.minimum(jnp.maximum(x, -3.0), 3.0)
        wv[sl] = _ALPHA / (1.0 + jnp.exp(-x))
      return carry
    lax.fori_loop(0, _ROWS, wrow, 0)

    for k in range(_NCHUNK):
      pltpu.sync_copy(
          a_hbm.at[pl.ds(k * _CHUNK_ELEMS + s * _SLICE, _SLICE)],
          shared.at[pl.ds(s * _SLICE, _SLICE)])
      plsc.subcore_barrier()

      def srow(j, carry):
        for q in range(8):
          sl = pl.ds(j * 128 + q * 16, 16)
          in_chunk = (flatv[sl] >> _CHUNK_SHIFT) == k
          upd2[j, pl.ds(q * 16, 16)] = jnp.where(in_chunk, wv[sl], 0.0)
        pltpu.async_copy(upd2.at[j], shared.at[idx2.at[j]], sem, add=True)
        return carry
      lax.fori_loop(0, _ROWS, srow, 0)

      def sdrain(j, carry):
        pltpu.make_async_copy(upd2.at[0], shared.at[idx2.at[0]], sem).wait()
        return carry
      lax.fori_loop(0, _ROWS, sdrain, 0)
      plsc.subcore_barrier()

      pltpu.sync_copy(
          shared.at[pl.ds(s * _SLICE, _SLICE)],
          out_hbm.at[b, pl.ds(k * _CHUNK_ELEMS + s * _SLICE, _SLICE)])
      plsc.subcore_barrier()


@jax.jit
def _dyn_adj(a_flat, src, dst, gates):
  mesh = plsc.VectorSubcoreMesh(core_axis_name="c", subcore_axis_name="s")
  f = functools.partial(
      pl.kernel,
      out_type=jax.ShapeDtypeStruct((_B, _N * _N), jnp.float32),
      mesh=mesh,
      scratch_types=[
          pltpu.VMEM((_EPT,), jnp.int32),
          pltpu.VMEM((_EPT,), jnp.int32),
          pltpu.VMEM((_EPT,), jnp.int32),
          pltpu.VMEM((_EPT,), jnp.float32),
          pltpu.VMEM((_EPT,), jnp.float32),
          pltpu.VMEM((_ROWS, 128), jnp.int32),
          pltpu.VMEM((_ROWS, 128), jnp.float32),
          pltpu.VMEM_SHARED((_CHUNK_ELEMS,), jnp.float32),
          pltpu.SemaphoreType.DMA,
      ],
  )(_body)
  return f(a_flat, src, dst, gates)


def kernel(A_base, edge_index, edge_gates):
  ei = edge_index.astype(jnp.int32)
  out = _dyn_adj(A_base.reshape(_N * _N), ei[0], ei[1], edge_gates)
  return out.reshape(_B, _N, _N)
